# baseline (speedup 1.0000x reference)
import functools

import jax
import jax.numpy as jnp
from jax import lax
from jax.experimental import pallas as pl
from jax.experimental.pallas import tpu as pltpu
from jax.experimental.pallas import tpu_sc as plsc

LANES = 16
P = 4


def _sc_kernel(x, pos_table, rows=None, full_out=False):
    seq_len, batch, d_model = x.shape
    if rows is None:
        rows = seq_len
    out_rows = seq_len if full_out else rows
    info = plsc.get_sparse_core_info()
    nw = info.num_cores * info.num_subcores
    spw = rows // nw
    nch = spw // P

    mesh = plsc.VectorSubcoreMesh(core_axis_name="c", subcore_axis_name="s")

    nslot = 4
    unroll = 2

    @functools.partial(
        pl.kernel,
        mesh=mesh,
        out_type=jax.ShapeDtypeStruct((out_rows, batch, d_model), x.dtype),
        scratch_types=[
            pltpu.VMEM((nslot, P, batch, d_model), jnp.float32),
            pltpu.VMEM((nslot, P, d_model), jnp.float32),
        ]
        + [pltpu.SemaphoreType.DMA] * (3 * nslot),
    )
    def sc_add(x_hbm, pos_hbm, out_hbm, xin, pin, *sems):
        wid = lax.axis_index("s") * info.num_cores + lax.axis_index("c")
        base = wid * spw
        sx = sems[0:nslot]
        sp = sems[nslot:2 * nslot]
        so = sems[2 * nslot:3 * nslot]

        def issue_in(i):
            s = i % nslot
            s0 = base + i * P
            cx = pltpu.async_copy(x_hbm.at[pl.ds(s0, P)], xin.at[s], sx[s])
            cp = pltpu.async_copy(pos_hbm.at[pl.ds(s0, P)], pin.at[s], sp[s])
            return cx, cp

        handles_in = {0: issue_in(0), 1: issue_in(1)}
        handles_out = {}
        for i in range(nch):
            s = i % nslot
            cx, cp = handles_in.pop(i)
            cx.wait()
            cp.wait()
            if i + 2 < nch:
                if i - 2 >= 0:
                    handles_out.pop(i - 2).wait()
                handles_in[i + 2] = issue_in(i + 2)

            def body(j, _):
                for u in range(unroll):
                    off = (j * unroll + u) * LANES
                    for p in range(P):
                        pv = pin[s, p, pl.ds(off, LANES)]
                        for b in range(batch):
                            plsc.addupdate(
                                xin.at[s, p, b, pl.ds(off, LANES)], pv
                            )
                return 0

            lax.fori_loop(0, d_model // (LANES * unroll), body, 0)
            s0 = base + i * P
            handles_out[i] = pltpu.async_copy(
                xin.at[s], out_hbm.at[pl.ds(s0, P)], so[s]
            )
        for i in sorted(handles_out):
            handles_out.pop(i).wait()

    return sc_add(x, pos_table[:seq_len])


S_BLK = 512


def _tc_body(x_ref, pos_ref, out_ref):
    pos = pos_ref[...]
    for b in range(x_ref.shape[1]):
        out_ref[:, b, :] = x_ref[:, b, :] + pos


def _tc_kernel(x, pos_table, start=0, blk=S_BLK):
    seq_len, batch, d_model = x.shape
    off = start // blk
    grid = ((seq_len - start) // blk,)
    return pl.pallas_call(
        _tc_body,
        grid=grid,
        in_specs=[
            pl.BlockSpec((blk, batch, d_model), lambda i: (i + off, 0, 0)),
            pl.BlockSpec((blk, d_model), lambda i: (i + off, 0)),
        ],
        out_specs=pl.BlockSpec((blk, batch, d_model), lambda i: (i + off, 0, 0)),
        out_shape=jax.ShapeDtypeStruct((seq_len, batch, d_model), x.dtype),
        compiler_params=pltpu.CompilerParams(
            dimension_semantics=("arbitrary",),
        ),
    )(x, pos_table[:seq_len])


def _tc_fill_body(prev_ref, x_ref, pos_ref, out_ref):
    del prev_ref
    pos = pos_ref[...]
    for b in range(x_ref.shape[1]):
        out_ref[:, b, :] = x_ref[:, b, :] + pos


def _tc_fill(prev, x, pos_table, start, blk):
    seq_len, batch, d_model = x.shape
    off = start // blk
    grid = ((seq_len - start) // blk,)
    return pl.pallas_call(
        _tc_fill_body,
        grid=grid,
        in_specs=[
            pl.BlockSpec(memory_space=pl.ANY),
            pl.BlockSpec((blk, batch, d_model), lambda i: (i + off, 0, 0)),
            pl.BlockSpec((blk, d_model), lambda i: (i + off, 0)),
        ],
        out_specs=pl.BlockSpec((blk, batch, d_model), lambda i: (i + off, 0, 0)),
        out_shape=jax.ShapeDtypeStruct((seq_len, batch, d_model), x.dtype),
        input_output_aliases={0: 0},
        compiler_params=pltpu.CompilerParams(
            dimension_semantics=("arbitrary",),
        ),
    )(prev, x, pos_table[:seq_len])


SC_ROWS = 256


def kernel(x, pos_table):
    sc_full = _sc_kernel(x, pos_table, rows=SC_ROWS, full_out=True)
    return _tc_fill(sc_full, x, pos_table, start=SC_ROWS, blk=256)


# speedup vs baseline: 1.0423x; 1.0423x over previous
import functools

import jax
import jax.numpy as jnp
from jax import lax
from jax.experimental import pallas as pl
from jax.experimental.pallas import tpu as pltpu
from jax.experimental.pallas import tpu_sc as plsc

LANES = 16
P = 4


def _sc_kernel(x, pos_table, rows=None):
    seq_len, batch, d_model = x.shape
    if rows is None:
        rows = seq_len
    info = plsc.get_sparse_core_info()
    nw = info.num_cores * info.num_subcores
    spw = rows // nw
    nch = spw // P

    mesh = plsc.VectorSubcoreMesh(core_axis_name="c", subcore_axis_name="s")

    nslot = 4
    unroll = 2

    @functools.partial(
        pl.kernel,
        mesh=mesh,
        out_type=jax.ShapeDtypeStruct((rows, batch, d_model), x.dtype),
        scratch_types=[
            pltpu.VMEM((nslot, P, batch, d_model), jnp.float32),
            pltpu.VMEM((nslot, P, d_model), jnp.float32),
        ]
        + [pltpu.SemaphoreType.DMA] * (3 * nslot),
    )
    def sc_add(x_hbm, pos_hbm, out_hbm, xin, pin, *sems):
        wid = lax.axis_index("s") * info.num_cores + lax.axis_index("c")
        base = wid * spw
        sx = sems[0:nslot]
        sp = sems[nslot:2 * nslot]
        so = sems[2 * nslot:3 * nslot]

        def issue_in(i):
            s = i % nslot
            s0 = base + i * P
            cx = pltpu.async_copy(x_hbm.at[pl.ds(s0, P)], xin.at[s], sx[s])
            cp = pltpu.async_copy(pos_hbm.at[pl.ds(s0, P)], pin.at[s], sp[s])
            return cx, cp

        handles_in = {0: issue_in(0)}
        if nch > 1:
            handles_in[1] = issue_in(1)
        handles_out = {}
        for i in range(nch):
            s = i % nslot
            cx, cp = handles_in.pop(i)
            cx.wait()
            cp.wait()
            if i + 2 < nch:
                if i - 2 >= 0:
                    handles_out.pop(i - 2).wait()
                handles_in[i + 2] = issue_in(i + 2)

            def body(j, _):
                for u in range(unroll):
                    off = (j * unroll + u) * LANES
                    for p in range(P):
                        pv = pin[s, p, pl.ds(off, LANES)]
                        for b in range(batch):
                            plsc.addupdate(
                                xin.at[s, p, b, pl.ds(off, LANES)], pv
                            )
                return 0

            lax.fori_loop(0, d_model // (LANES * unroll), body, 0)
            s0 = base + i * P
            handles_out[i] = pltpu.async_copy(
                xin.at[s], out_hbm.at[pl.ds(s0, P)], so[s]
            )
        for i in sorted(handles_out):
            handles_out.pop(i).wait()

    return sc_add(x, pos_table[:seq_len])


def _tc_body(x_ref, pos_ref, out_ref):
    pos = pos_ref[...]
    for b in range(x_ref.shape[1]):
        out_ref[:, b, :] = x_ref[:, b, :] + pos


def _tc_kernel(x, pos_table, start=0, blk=256):
    seq_len, batch, d_model = x.shape
    off = start // blk
    grid = ((seq_len - start) // blk,)
    return pl.pallas_call(
        _tc_body,
        grid=grid,
        in_specs=[
            pl.BlockSpec((blk, batch, d_model), lambda i: (i + off, 0, 0)),
            pl.BlockSpec((blk, d_model), lambda i: (i + off, 0)),
        ],
        out_specs=pl.BlockSpec((blk, batch, d_model), lambda i: (i + off, 0, 0)),
        out_shape=jax.ShapeDtypeStruct((seq_len, batch, d_model), x.dtype),
        compiler_params=pltpu.CompilerParams(
            dimension_semantics=("arbitrary",),
        ),
    )(x, pos_table[:seq_len])


SC_ROWS = 256


def kernel(x, pos_table):
    sc_part = _sc_kernel(x, pos_table, rows=SC_ROWS)
    tc_full = _tc_kernel(x, pos_table, start=SC_ROWS, blk=256)
    return lax.dynamic_update_slice(tc_full, sc_part, (0, 0, 0))


# speedup vs baseline: 1.0543x; 1.0115x over previous
import functools

import jax
import jax.numpy as jnp
from jax import lax
from jax.experimental import pallas as pl
from jax.experimental.pallas import tpu as pltpu
from jax.experimental.pallas import tpu_sc as plsc

LANES = 16
P = 4


def _sc_kernel(x, pos_table, rows=None):
    seq_len, batch, d_model = x.shape
    if rows is None:
        rows = seq_len
    info = plsc.get_sparse_core_info()
    nw = info.num_cores * info.num_subcores
    spw = rows // nw
    nch = spw // P

    mesh = plsc.VectorSubcoreMesh(core_axis_name="c", subcore_axis_name="s")

    nslot = 4
    unroll = 2

    @functools.partial(
        pl.kernel,
        mesh=mesh,
        out_type=jax.ShapeDtypeStruct((rows, batch, d_model), x.dtype),
        scratch_types=[
            pltpu.VMEM((nslot, P, batch, d_model), jnp.float32),
            pltpu.VMEM((nslot, P, d_model), jnp.float32),
        ]
        + [pltpu.SemaphoreType.DMA] * (3 * nslot),
    )
    def sc_add(x_hbm, pos_hbm, out_hbm, xin, pin, *sems):
        wid = lax.axis_index("s") * info.num_cores + lax.axis_index("c")
        base = wid * spw
        sx = sems[0:nslot]
        sp = sems[nslot:2 * nslot]
        so = sems[2 * nslot:3 * nslot]

        def issue_in(i):
            s = i % nslot
            s0 = base + i * P
            cx = pltpu.async_copy(x_hbm.at[pl.ds(s0, P)], xin.at[s], sx[s])
            cp = pltpu.async_copy(pos_hbm.at[pl.ds(s0, P)], pin.at[s], sp[s])
            return cx, cp

        handles_in = {0: issue_in(0)}
        if nch > 1:
            handles_in[1] = issue_in(1)
        handles_out = {}
        for i in range(nch):
            s = i % nslot
            cx, cp = handles_in.pop(i)
            cx.wait()
            cp.wait()
            if i + 2 < nch:
                if i - 2 >= 0:
                    handles_out.pop(i - 2).wait()
                handles_in[i + 2] = issue_in(i + 2)

            def body(j, _):
                for u in range(unroll):
                    off = (j * unroll + u) * LANES
                    for p in range(P):
                        pv = pin[s, p, pl.ds(off, LANES)]
                        for b in range(batch):
                            plsc.addupdate(
                                xin.at[s, p, b, pl.ds(off, LANES)], pv
                            )
                return 0

            lax.fori_loop(0, d_model // (LANES * unroll), body, 0)
            s0 = base + i * P
            handles_out[i] = pltpu.async_copy(
                xin.at[s], out_hbm.at[pl.ds(s0, P)], so[s]
            )
        for i in sorted(handles_out):
            handles_out.pop(i).wait()

    return sc_add(x, pos_table[:seq_len])


def _tc_body(x_ref, pos_ref, out_ref):
    pos = pos_ref[...]
    for b in range(x_ref.shape[1]):
        out_ref[:, b, :] = x_ref[:, b, :] + pos


def _tc_kernel(x, pos_table, start=0, blk=256):
    seq_len, batch, d_model = x.shape
    off = start // blk
    grid = ((seq_len - start) // blk,)
    return pl.pallas_call(
        _tc_body,
        grid=grid,
        in_specs=[
            pl.BlockSpec((blk, batch, d_model), lambda i: (i + off, 0, 0)),
            pl.BlockSpec((blk, d_model), lambda i: (i + off, 0)),
        ],
        out_specs=pl.BlockSpec((blk, batch, d_model), lambda i: (i + off, 0, 0)),
        out_shape=jax.ShapeDtypeStruct((seq_len, batch, d_model), x.dtype),
        compiler_params=pltpu.CompilerParams(
            dimension_semantics=("arbitrary",),
        ),
    )(x, pos_table[:seq_len])


SC_ROWS = 128


def kernel(x, pos_table):
    sc_part = _sc_kernel(x, pos_table, rows=SC_ROWS)
    tc_full = _tc_kernel(x, pos_table, start=SC_ROWS, blk=128)
    return lax.dynamic_update_slice(tc_full, sc_part, (0, 0, 0))
